# baseline (speedup 1.0000x reference)
import functools

import jax
import jax.numpy as jnp
from jax import lax
from jax.experimental import pallas as pl
from jax.experimental.pallas import tpu as pltpu
from jax.experimental.pallas import tpu_sc as plsc

N = 10000
D = 128
NC = 2
NS = 16
NW = NC * NS
CHUNK = 56
C2 = 192
EP = NW * C2 * CHUNK
RB = 24
NBATCH = C2 // RB
SLOTS = 6
ACC_ROWS = 10240
ZROWS_PER_TILE = ACC_ROWS // NS
DUMP_LO = N + 8
ZR = 128

R = 2000
GRID = N // R


def _seg_sum_body(m, eidx, zrows, out, sr0, dr0,
                  gb0, gb1, gb2, gb3, gb4, gb5, acc,
                  g0, g1, g2, g3, g4, g5, s0, s1, s2, s3, s4, s5, i0):
    c = lax.axis_index("c")
    s = lax.axis_index("s")
    w = c * NS + s
    gbs = (gb0, gb1, gb2, gb3, gb4, gb5)
    gsem = (g0, g1, g2, g3, g4, g5)
    ssem = (s0, s1, s2, s3, s4, s5)

    def fire_idx(bb):
        sl = pl.ds(bb * RB, RB)
        pltpu.async_copy(eidx.at[0].at[w].at[sl], sr0, i0)
        pltpu.async_copy(eidx.at[1].at[w].at[sl], dr0, i0)

    def wait_idx():
        sl = pl.ds(0, RB)
        pltpu.make_async_copy(eidx.at[0].at[w].at[sl], sr0, i0).wait()
        pltpu.make_async_copy(eidx.at[1].at[w].at[sl], dr0, i0).wait()

    fire_idx(0)
    zbase = s * ZROWS_PER_TILE
    for k in range(ZROWS_PER_TILE // ZR):
        pltpu.sync_copy(zrows, acc.at[pl.ds(zbase + k * ZR, ZR)])
    plsc.subcore_barrier()

    def wait_g(q):
        pltpu.make_async_copy(m.at[sr0.at[0]], gbs[q], gsem[q]).wait()

    def wait_s(q):
        pltpu.make_async_copy(gbs[q], acc.at[dr0.at[0]], ssem[q]).wait()

    def fire_g(ch, q):
        pltpu.async_copy(m.at[sr0.at[ch]], gbs[q], gsem[q])

    def fire_s(ch, q):
        pltpu.async_copy(gbs[q], acc.at[dr0.at[ch]], ssem[q], add=True)

    for bb in range(NBATCH):
        wait_idx()
        fire_g(0, 0)
        fire_g(1, 1)
        for ch in (0, 1, 2, 3):
            wait_g(ch)
            fire_s(ch, ch)
            fire_g(ch + 2, ch + 2)

        def body(jj, carry):
            base = 6 * jj + 4
            for u in range(6):
                ch = base + u
                q = (4 + u) % 6
                nq = u % 6
                wait_g(q)
                fire_s(ch, q)
                wait_s(nq)
                fire_g(ch + 2, nq)
            return carry

        lax.fori_loop(0, (RB - 6) // 6, body, 0)
        for ch in (RB - 2, RB - 1):
            q = ch % 6
            wait_g(q)
            fire_s(ch, q)
        for q in range(SLOTS):
            wait_s(q)
        if bb + 1 < NBATCH:
            fire_idx(bb + 1)

    plsc.subcore_barrier()
    orows = pl.ds(s * ZROWS_PER_TILE, ZROWS_PER_TILE)
    pltpu.sync_copy(acc.at[orows], out.at[c].at[orows])


_seg_sum = functools.partial(
    pl.kernel,
    out_type=jax.ShapeDtypeStruct((NC, ACC_ROWS, D), jnp.float32),
    mesh=plsc.VectorSubcoreMesh(core_axis_name="c", subcore_axis_name="s",
                                num_cores=NC, num_subcores=NS),
    scratch_types=[
        pltpu.VMEM((RB, CHUNK), jnp.int32),
        pltpu.VMEM((RB, CHUNK), jnp.int32),
        pltpu.VMEM((CHUNK, D), jnp.float32),
        pltpu.VMEM((CHUNK, D), jnp.float32),
        pltpu.VMEM((CHUNK, D), jnp.float32),
        pltpu.VMEM((CHUNK, D), jnp.float32),
        pltpu.VMEM((CHUNK, D), jnp.float32),
        pltpu.VMEM((CHUNK, D), jnp.float32),
        pltpu.VMEM_SHARED((ACC_ROWS, D), jnp.float32),
        pltpu.SemaphoreType.DMA,
        pltpu.SemaphoreType.DMA,
        pltpu.SemaphoreType.DMA,
        pltpu.SemaphoreType.DMA,
        pltpu.SemaphoreType.DMA,
        pltpu.SemaphoreType.DMA,
        pltpu.SemaphoreType.DMA,
        pltpu.SemaphoreType.DMA,
        pltpu.SemaphoreType.DMA,
        pltpu.SemaphoreType.DMA,
        pltpu.SemaphoreType.DMA,
        pltpu.SemaphoreType.DMA,
        pltpu.SemaphoreType.DMA,
    ],
)(_seg_sum_body)


def _w_spec():
    return pl.BlockSpec((D, D), lambda i: (0, 0))


def _b_spec():
    return pl.BlockSpec((1, D), lambda i: (0, 0))


def _gru_first():
    def body(x, Wz, Wh, Wm, bz, bh, h_out, m_out):
        a = x[0]
        z = jax.nn.sigmoid(jnp.dot(a, Wz[...]) + bz[...])
        hc = jnp.tanh(jnp.dot(a, Wh[...]) + bh[...])
        h = z * hc
        h_out[...] = h
        m_out[...] = jnp.dot(h, Wm[...])

    return pl.pallas_call(
        body,
        grid=(GRID,),
        in_specs=[
            pl.BlockSpec((1, R, D), lambda i: (0, i, 0)),
            _w_spec(), _w_spec(), _w_spec(), _b_spec(), _b_spec(),
        ],
        out_specs=[
            pl.BlockSpec((R, D), lambda i: (i, 0)),
            pl.BlockSpec((R, D), lambda i: (i, 0)),
        ],
        out_shape=[
            jax.ShapeDtypeStruct((N, D), jnp.float32),
            jax.ShapeDtypeStruct((N, D), jnp.float32),
        ],
    )


def _gru_step(t, emit_m):
    def body(x, agg, h_in, Wz, Uz, Wr, Ur, Wh, Uh, Wm, bz, br, bh,
             h_out, *maybe_m):
        a = x[0] + agg[0] + agg[1]
        h = h_in[...]
        z = jax.nn.sigmoid(jnp.dot(a, Wz[...]) + jnp.dot(h, Uz[...]) + bz[...])
        r = jax.nn.sigmoid(jnp.dot(a, Wr[...]) + jnp.dot(h, Ur[...]) + br[...])
        hc = jnp.tanh(jnp.dot(a, Wh[...]) + jnp.dot(r * h, Uh[...]) + bh[...])
        hn = (1.0 - z) * h + z * hc
        h_out[...] = hn
        if maybe_m:
            maybe_m[0][...] = jnp.dot(hn, Wm[...])

    out_specs = [pl.BlockSpec((R, D), lambda i: (i, 0))]
    out_shape = [jax.ShapeDtypeStruct((N, D), jnp.float32)]
    if emit_m:
        out_specs.append(pl.BlockSpec((R, D), lambda i: (i, 0)))
        out_shape.append(jax.ShapeDtypeStruct((N, D), jnp.float32))

    return pl.pallas_call(
        body,
        grid=(GRID,),
        in_specs=[
            pl.BlockSpec((1, R, D), lambda i, t=t: (t, i, 0)),
            pl.BlockSpec((NC, R, D), lambda i: (0, i, 0)),
            pl.BlockSpec((R, D), lambda i: (i, 0)),
            _w_spec(), _w_spec(), _w_spec(), _w_spec(), _w_spec(), _w_spec(),
            _w_spec(), _b_spec(), _b_spec(), _b_spec(),
        ],
        out_specs=out_specs,
        out_shape=out_shape,
    )


def kernel(x, edge_index, W_msg, Wz, Uz, bz, Wr, Ur, br, Wh, Uh, bh):
    T = x.shape[0]
    src = edge_index[0]
    dst = edge_index[1]
    e_per_tile = src.shape[0] // NW
    pad_per_tile = C2 * CHUNK - e_per_tile
    pad2 = jnp.zeros((NW, pad_per_tile), jnp.int32)
    src_p = jnp.concatenate([src.reshape(NW, e_per_tile), pad2], axis=1)
    dump = DUMP_LO + (jnp.arange(pad_per_tile, dtype=jnp.int32)
                      % (ACC_ROWS - DUMP_LO))
    dst_p = jnp.concatenate(
        [dst.reshape(NW, e_per_tile),
         jnp.broadcast_to(dump, (NW, pad_per_tile))], axis=1)
    eidx = jnp.stack([src_p, dst_p]).reshape(2, NW, C2, CHUNK)
    zrows = jnp.zeros((ZR, D), jnp.float32)
    bz2 = bz.reshape(1, D)
    br2 = br.reshape(1, D)
    bh2 = bh.reshape(1, D)

    h, m = _gru_first()(x, Wz, Wh, W_msg, bz2, bh2)
    for t in range(1, T):
        agg = _seg_sum(m, eidx, zrows)
        outs = _gru_step(t, emit_m=(t < T - 1))(
            x, agg, h, Wz, Uz, Wr, Ur, Wh, Uh, W_msg, bz2, br2, bh2)
        if t < T - 1:
            h, m = outs
        else:
            h = outs[0]
    return h


# speedup vs baseline: 2.3991x; 2.3991x over previous
import functools

import jax
import jax.numpy as jnp
from jax import lax
from jax.experimental import pallas as pl
from jax.experimental.pallas import tpu as pltpu
from jax.experimental.pallas import tpu_sc as plsc

N = 10000
D = 128
NC = 2
NS = 16
NW = NC * NS
CHUNK = 64
C2 = 160
EP = NW * C2 * CHUNK
RB = 32
NBATCH = C2 // RB
SLOTS = 5
ACC_ROWS = 10240
ZROWS_PER_TILE = ACC_ROWS // NS
DUMP_LO = N + 8
ZR = 128

R = 2000
GRID = N // R


def _seg_sum_body(m, eidx, zrows, out, sr0, dr0,
                  gb0, gb1, gb2, gb3, gb4, acc,
                  g0, g1, g2, g3, g4, s0, s1, s2, s3, s4, i0):
    c = lax.axis_index("c")
    s = lax.axis_index("s")
    w = c * NS + s
    gbs = (gb0, gb1, gb2, gb3, gb4)
    gsem = (g0, g1, g2, g3, g4)
    ssem = (s0, s1, s2, s3, s4)

    def fire_idx(bb):
        sl = pl.ds(bb * RB, RB)
        pltpu.async_copy(eidx.at[0].at[w].at[sl], sr0, i0)
        pltpu.async_copy(eidx.at[1].at[w].at[sl], dr0, i0)

    def wait_idx():
        sl = pl.ds(0, RB)
        pltpu.make_async_copy(eidx.at[0].at[w].at[sl], sr0, i0).wait()
        pltpu.make_async_copy(eidx.at[1].at[w].at[sl], dr0, i0).wait()

    fire_idx(0)
    zbase = s * ZROWS_PER_TILE
    for k in range(ZROWS_PER_TILE // ZR):
        pltpu.sync_copy(zrows, acc.at[pl.ds(zbase + k * ZR, ZR)])
    plsc.subcore_barrier()

    def wait_g(q):
        pltpu.make_async_copy(m.at[sr0.at[0]], gbs[q], gsem[q]).wait()

    def wait_s(q):
        pltpu.make_async_copy(gbs[q], acc.at[dr0.at[0]], ssem[q]).wait()

    def fire_g(ch, q):
        pltpu.async_copy(m.at[sr0.at[ch]], gbs[q], gsem[q])

    def fire_s(ch, q):
        pltpu.async_copy(gbs[q], acc.at[dr0.at[ch]], ssem[q], add=True)

    for bb in range(NBATCH):
        wait_idx()
        fire_g(0, 0)
        fire_g(1, 1)
        for ch in (0, 1, 2):
            wait_g(ch)
            fire_s(ch, ch)
            fire_g(ch + 2, ch + 2)

        def body(jj, carry):
            base = 5 * jj + 3
            for u in range(5):
                ch = base + u
                q = (3 + u) % 5
                nq = u % 5
                wait_g(q)
                fire_s(ch, q)
                wait_s(nq)
                fire_g(ch + 2, nq)
            return carry

        lax.fori_loop(0, (RB - 7) // 5, body, 0)
        for ch in range(3 + 5 * ((RB - 7) // 5), RB - 2):
            q = ch % 5
            nq = (ch + 2) % 5
            wait_g(q)
            fire_s(ch, q)
            wait_s(nq)
            fire_g(ch + 2, nq)
        for ch in (RB - 2, RB - 1):
            q = ch % 5
            wait_g(q)
            fire_s(ch, q)
        for q in range(SLOTS):
            wait_s(q)
        if bb + 1 < NBATCH:
            fire_idx(bb + 1)

    plsc.subcore_barrier()
    orows = pl.ds(s * ZROWS_PER_TILE, ZROWS_PER_TILE)
    pltpu.sync_copy(acc.at[orows], out.at[c].at[orows])


_seg_sum = functools.partial(
    pl.kernel,
    out_type=jax.ShapeDtypeStruct((NC, ACC_ROWS, D), jnp.float32),
    mesh=plsc.VectorSubcoreMesh(core_axis_name="c", subcore_axis_name="s",
                                num_cores=NC, num_subcores=NS),
    scratch_types=[
        pltpu.VMEM((RB, CHUNK), jnp.int32),
        pltpu.VMEM((RB, CHUNK), jnp.int32),
        pltpu.VMEM((CHUNK, D), jnp.float32),
        pltpu.VMEM((CHUNK, D), jnp.float32),
        pltpu.VMEM((CHUNK, D), jnp.float32),
        pltpu.VMEM((CHUNK, D), jnp.float32),
        pltpu.VMEM((CHUNK, D), jnp.float32),
        pltpu.VMEM_SHARED((ACC_ROWS, D), jnp.float32),
        pltpu.SemaphoreType.DMA,
        pltpu.SemaphoreType.DMA,
        pltpu.SemaphoreType.DMA,
        pltpu.SemaphoreType.DMA,
        pltpu.SemaphoreType.DMA,
        pltpu.SemaphoreType.DMA,
        pltpu.SemaphoreType.DMA,
        pltpu.SemaphoreType.DMA,
        pltpu.SemaphoreType.DMA,
        pltpu.SemaphoreType.DMA,
        pltpu.SemaphoreType.DMA,
    ],
)(_seg_sum_body)


def _w_spec():
    return pl.BlockSpec((D, D), lambda i: (0, 0))


def _b_spec():
    return pl.BlockSpec((1, D), lambda i: (0, 0))


def _gru_first():
    def body(x, Wz, Wh, Wm, bz, bh, h_out, m_out):
        a = x[0]
        z = jax.nn.sigmoid(jnp.dot(a, Wz[...]) + bz[...])
        hc = jnp.tanh(jnp.dot(a, Wh[...]) + bh[...])
        h = z * hc
        h_out[...] = h
        m_out[...] = jnp.dot(h, Wm[...])

    return pl.pallas_call(
        body,
        grid=(GRID,),
        in_specs=[
            pl.BlockSpec((1, R, D), lambda i: (0, i, 0)),
            _w_spec(), _w_spec(), _w_spec(), _b_spec(), _b_spec(),
        ],
        out_specs=[
            pl.BlockSpec((R, D), lambda i: (i, 0)),
            pl.BlockSpec((R, D), lambda i: (i, 0)),
        ],
        out_shape=[
            jax.ShapeDtypeStruct((N, D), jnp.float32),
            jax.ShapeDtypeStruct((N, D), jnp.float32),
        ],
    )


def _gru_step(t, emit_m):
    def body(x, agg, h_in, Wz, Uz, Wr, Ur, Wh, Uh, Wm, bz, br, bh,
             h_out, *maybe_m):
        a = x[0] + agg[0] + agg[1]
        h = h_in[...]
        z = jax.nn.sigmoid(jnp.dot(a, Wz[...]) + jnp.dot(h, Uz[...]) + bz[...])
        r = jax.nn.sigmoid(jnp.dot(a, Wr[...]) + jnp.dot(h, Ur[...]) + br[...])
        hc = jnp.tanh(jnp.dot(a, Wh[...]) + jnp.dot(r * h, Uh[...]) + bh[...])
        hn = (1.0 - z) * h + z * hc
        h_out[...] = hn
        if maybe_m:
            maybe_m[0][...] = jnp.dot(hn, Wm[...])

    out_specs = [pl.BlockSpec((R, D), lambda i: (i, 0))]
    out_shape = [jax.ShapeDtypeStruct((N, D), jnp.float32)]
    if emit_m:
        out_specs.append(pl.BlockSpec((R, D), lambda i: (i, 0)))
        out_shape.append(jax.ShapeDtypeStruct((N, D), jnp.float32))

    return pl.pallas_call(
        body,
        grid=(GRID,),
        in_specs=[
            pl.BlockSpec((1, R, D), lambda i, t=t: (t, i, 0)),
            pl.BlockSpec((NC, R, D), lambda i: (0, i, 0)),
            pl.BlockSpec((R, D), lambda i: (i, 0)),
            _w_spec(), _w_spec(), _w_spec(), _w_spec(), _w_spec(), _w_spec(),
            _w_spec(), _b_spec(), _b_spec(), _b_spec(),
        ],
        out_specs=out_specs,
        out_shape=out_shape,
    )


def kernel(x, edge_index, W_msg, Wz, Uz, bz, Wr, Ur, br, Wh, Uh, bh):
    T = x.shape[0]
    src = edge_index[0]
    dst = edge_index[1]
    e_per_tile = src.shape[0] // NW
    pad_per_tile = C2 * CHUNK - e_per_tile
    pad2 = jnp.zeros((NW, pad_per_tile), jnp.int32)
    src_p = jnp.concatenate([src.reshape(NW, e_per_tile), pad2], axis=1)
    dump = DUMP_LO + (jnp.arange(pad_per_tile, dtype=jnp.int32)
                      % (ACC_ROWS - DUMP_LO))
    dst_p = jnp.concatenate(
        [dst.reshape(NW, e_per_tile),
         jnp.broadcast_to(dump, (NW, pad_per_tile))], axis=1)
    eidx = jnp.stack([src_p, dst_p]).reshape(2, NW, C2, CHUNK)
    zrows = jnp.zeros((ZR, D), jnp.float32)
    bz2 = bz.reshape(1, D)
    br2 = br.reshape(1, D)
    bh2 = bh.reshape(1, D)

    h, m = _gru_first()(x, Wz, Wh, W_msg, bz2, bh2)
    for t in range(1, T):
        agg = _seg_sum(m, eidx, zrows)
        outs = _gru_step(t, emit_m=(t < T - 1))(
            x, agg, h, Wz, Uz, Wr, Ur, Wh, Uh, W_msg, bz2, br2, bh2)
        if t < T - 1:
            h, m = outs
        else:
            h = outs[0]
    return h
